# baseline (speedup 1.0000x reference)
import math

import jax
import jax.numpy as jnp
import numpy as np
from jax.experimental import pallas as pl
from jax.experimental.pallas import tpu as pltpu
from jax.experimental.shard_map import shard_map

L = 10
H = L * L
B = 49152
S = B // 128


def _sh_block(pl_ref, out_ref):
    lon = pl_ref[0:1, :].reshape(S, 128)
    lat = pl_ref[1:2, :].reshape(S, 128)
    rad = math.pi / 180.0
    phi = (lon + 180.0) * rad
    theta = (lat + 90.0) * rad
    x = jnp.cos(theta)
    sx = jnp.sin(theta)

    c = [jnp.ones_like(phi), jnp.cos(phi)]
    s = [jnp.zeros_like(phi), jnp.sin(phi)]
    two_c1 = 2.0 * c[1]
    for m in range(2, L):
        c.append(two_c1 * c[m - 1] - c[m - 2])
        s.append(two_c1 * s[m - 1] - s[m - 2])

    P = {(0, 0): jnp.ones_like(x)}
    for m in range(1, L):
        df = 1.0
        for k in range(1, 2 * m, 2):
            df *= float(k)
        P[(m, m)] = ((-1.0) ** m) * df * (sx ** m)
    for m in range(0, L - 1):
        P[(m + 1, m)] = (2.0 * m + 1.0) * x * P[(m, m)]
    for m in range(0, L):
        for l in range(m + 2, L):
            a = (2.0 * l - 1.0) / float(l - m)
            b = (l + m - 1.0) / float(l - m)
            P[(l, m)] = a * x * P[(l - 1, m)] - b * P[(l - 2, m)]

    ys = []
    for l in range(L):
        for m in range(-l, l + 1):
            am = abs(m)
            K = math.sqrt((2.0 * l + 1.0) / (4.0 * math.pi)
                          * math.factorial(l - am) / math.factorial(l + am))
            if m > 0:
                ys.append((math.sqrt(2.0) * K) * (c[m] * P[(l, am)]))
            elif m < 0:
                ys.append((math.sqrt(2.0) * K) * (s[am] * P[(l, am)]))
            else:
                ys.append(K * P[(l, 0)])

    out_ref[...] = jnp.stack(ys, axis=0).reshape(H, B)


def _sh_pallas(lonlat):
    n = lonlat.shape[0]
    g = -(-n // B)
    out_t = pl.pallas_call(
        _sh_block,
        grid=(g,),
        in_specs=[
            pl.BlockSpec((2, B), lambda i: (0, i)),
        ],
        out_specs=pl.BlockSpec((H, B), lambda i: (0, i)),
        out_shape=jax.ShapeDtypeStruct((H, n), jnp.float32),
        compiler_params=pltpu.CompilerParams(
            dimension_semantics=("parallel",),
        ),
    )(lonlat.T)
    return out_t.T


def kernel(lonlat):
    return _sh_pallas(lonlat)


# speedup vs baseline: 1.0119x; 1.0119x over previous
import math

import jax
import jax.numpy as jnp
import numpy as np
from jax.experimental import pallas as pl
from jax.experimental.pallas import tpu as pltpu
from jax.experimental.shard_map import shard_map

L = 10
H = L * L
B = 16384
S = B // 128


def _sh_block(pl_ref, out_ref):
    lon = pl_ref[0:1, :].reshape(S, 128)
    lat = pl_ref[1:2, :].reshape(S, 128)
    rad = math.pi / 180.0
    phi = (lon + 180.0) * rad
    theta = (lat + 90.0) * rad
    x = jnp.cos(theta)
    sx = jnp.sin(theta)

    c = [jnp.ones_like(phi), jnp.cos(phi)]
    s = [jnp.zeros_like(phi), jnp.sin(phi)]
    two_c1 = 2.0 * c[1]
    for m in range(2, L):
        c.append(two_c1 * c[m - 1] - c[m - 2])
        s.append(two_c1 * s[m - 1] - s[m - 2])

    P = {(0, 0): jnp.ones_like(x)}
    for m in range(1, L):
        df = 1.0
        for k in range(1, 2 * m, 2):
            df *= float(k)
        P[(m, m)] = ((-1.0) ** m) * df * (sx ** m)
    for m in range(0, L - 1):
        P[(m + 1, m)] = (2.0 * m + 1.0) * x * P[(m, m)]
    for m in range(0, L):
        for l in range(m + 2, L):
            a = (2.0 * l - 1.0) / float(l - m)
            b = (l + m - 1.0) / float(l - m)
            P[(l, m)] = a * x * P[(l - 1, m)] - b * P[(l - 2, m)]

    ys = []
    for l in range(L):
        for m in range(-l, l + 1):
            am = abs(m)
            K = math.sqrt((2.0 * l + 1.0) / (4.0 * math.pi)
                          * math.factorial(l - am) / math.factorial(l + am))
            if m > 0:
                ys.append((math.sqrt(2.0) * K) * (c[m] * P[(l, am)]))
            elif m < 0:
                ys.append((math.sqrt(2.0) * K) * (s[am] * P[(l, am)]))
            else:
                ys.append(K * P[(l, 0)])

    out_ref[...] = jnp.stack(ys, axis=0).reshape(H, B)


def _sh_pallas(lonlat):
    n = lonlat.shape[0]
    g = -(-n // B)
    out_t = pl.pallas_call(
        _sh_block,
        grid=(g,),
        in_specs=[
            pl.BlockSpec((2, B), lambda i: (0, i)),
        ],
        out_specs=pl.BlockSpec((H, B), lambda i: (0, i)),
        out_shape=jax.ShapeDtypeStruct((H, n), jnp.float32),
        compiler_params=pltpu.CompilerParams(
            dimension_semantics=("parallel",),
        ),
    )(lonlat.T)
    return out_t.T


def kernel(lonlat):
    return _sh_pallas(lonlat)


# speedup vs baseline: 1.0272x; 1.0151x over previous
import math

import jax
import jax.numpy as jnp
from jax.experimental import pallas as pl
from jax.experimental.pallas import tpu as pltpu

L = 10
H = L * L
B = 32768
S = B // 128


def _sh_block(pl_ref, out_ref):
    lon = pl_ref[0:1, :].reshape(S, 128)
    lat = pl_ref[1:2, :].reshape(S, 128)
    rad = math.pi / 180.0
    phi = (lon + 180.0) * rad
    theta = (lat + 90.0) * rad
    x = jnp.cos(theta)
    sx = jnp.sin(theta)

    c = [jnp.ones_like(phi), jnp.cos(phi)]
    s = [jnp.zeros_like(phi), jnp.sin(phi)]
    two_c1 = 2.0 * c[1]
    for m in range(2, L):
        c.append(two_c1 * c[m - 1] - c[m - 2])
        s.append(two_c1 * s[m - 1] - s[m - 2])

    P = {(0, 0): jnp.ones_like(x)}
    for m in range(1, L):
        df = 1.0
        for k in range(1, 2 * m, 2):
            df *= float(k)
        P[(m, m)] = ((-1.0) ** m) * df * (sx ** m)
    for m in range(0, L - 1):
        P[(m + 1, m)] = (2.0 * m + 1.0) * x * P[(m, m)]
    for m in range(0, L):
        for l in range(m + 2, L):
            a = (2.0 * l - 1.0) / float(l - m)
            b = (l + m - 1.0) / float(l - m)
            P[(l, m)] = a * x * P[(l - 1, m)] - b * P[(l - 2, m)]

    ys = []
    for l in range(L):
        for m in range(-l, l + 1):
            am = abs(m)
            K = math.sqrt((2.0 * l + 1.0) / (4.0 * math.pi)
                          * math.factorial(l - am) / math.factorial(l + am))
            if m > 0:
                ys.append((math.sqrt(2.0) * K) * (c[m] * P[(l, am)]))
            elif m < 0:
                ys.append((math.sqrt(2.0) * K) * (s[am] * P[(l, am)]))
            else:
                ys.append(K * P[(l, 0)])

    out_ref[...] = jnp.stack(ys, axis=0).reshape(H, B)


def _sh_pallas(lonlat):
    n = lonlat.shape[0]
    g = -(-n // B)
    out_t = pl.pallas_call(
        _sh_block,
        grid=(g,),
        in_specs=[
            pl.BlockSpec((2, B), lambda i: (0, i)),
        ],
        out_specs=pl.BlockSpec((H, B), lambda i: (0, i)),
        out_shape=jax.ShapeDtypeStruct((H, n), jnp.float32),
        compiler_params=pltpu.CompilerParams(
            dimension_semantics=("parallel",),
        ),
    )(lonlat.T)
    return out_t.T


def kernel(lonlat):
    return _sh_pallas(lonlat)
